# Initial kernel scaffold; baseline (speedup 1.0000x reference)
import functools

import jax
import jax.numpy as jnp
from jax import lax
from jax.experimental import pallas as pl
from jax.experimental.pallas import tpu as pltpu
from jax.experimental.pallas import tpu_sc as plsc

ALPHA = 0.5
LANES = 16
NTILES = 16
CHUNK = 128


def _ceil_to(v, m):
    return -(-v // m) * m


@functools.lru_cache(maxsize=None)
def _make_deg_kernel(n_pad, e_pad):
    chunks = e_pad // (NTILES * CHUNK)
    rows_per_tile = n_pad // NTILES
    mesh = plsc.VectorSubcoreMesh(core_axis_name="c", subcore_axis_name="s")

    @functools.partial(
        pl.kernel,
        out_type=[jax.ShapeDtypeStruct((n_pad, LANES), jnp.float32),
                  jax.ShapeDtypeStruct((n_pad, LANES), jnp.float32)],
        mesh=mesh,
        scratch_types=[
            pltpu.VMEM((CHUNK,), jnp.int32),
            pltpu.VMEM((CHUNK, LANES), jnp.float32),
            pltpu.VMEM_SHARED((n_pad, LANES), jnp.float32),
        ],
    )
    def deg_kernel(zeros16, ones16, row_idx, col_idx, out_deg, in_deg,
                   idx_v, ones_v, acc):
        c = lax.axis_index("c")
        s = lax.axis_index("s")
        base = s * rows_per_tile
        pltpu.sync_copy(zeros16.at[pl.ds(base, rows_per_tile)],
                        acc.at[pl.ds(base, rows_per_tile)])
        pltpu.sync_copy(ones16, ones_v)
        plsc.subcore_barrier()

        def run(idx_hbm, out):
            ebase = s * (chunks * CHUNK)

            def body(i, carry):
                pltpu.sync_copy(idx_hbm.at[pl.ds(ebase + i * CHUNK, CHUNK)],
                                idx_v)
                pltpu.sync_copy(ones_v, acc.at[idx_v], add=True)
                return carry

            lax.fori_loop(0, chunks, body, 0)
            plsc.subcore_barrier()
            pltpu.sync_copy(acc.at[pl.ds(base, rows_per_tile)],
                            out.at[pl.ds(base, rows_per_tile)])

        @pl.when(c == 0)
        def _():
            run(row_idx, out_deg)

        @pl.when(c == 1)
        def _():
            run(col_idx, in_deg)

    return deg_kernel


@functools.lru_cache(maxsize=None)
def _make_agg_kernel(n_pad, e_pad, d):
    chunks = e_pad // (NTILES * CHUNK)
    rows_per_tile = n_pad // NTILES
    mesh = plsc.VectorSubcoreMesh(core_axis_name="c", subcore_axis_name="s")

    @functools.partial(
        pl.kernel,
        out_type=[jax.ShapeDtypeStruct((n_pad, d), jnp.float32),
                  jax.ShapeDtypeStruct((n_pad, d), jnp.float32)],
        mesh=mesh,
        scratch_types=[
            pltpu.VMEM((CHUNK,), jnp.int32),
            pltpu.VMEM((CHUNK,), jnp.int32),
            pltpu.VMEM((CHUNK, d), jnp.float32),
            pltpu.VMEM_SHARED((n_pad, d), jnp.float32),
            pltpu.SemaphoreType.DMA,
        ],
    )
    def agg_kernel(zeros, z_src, z_dst, row_idx, col_idx, out_src, out_dst,
                   idx_g, idx_s, rows_v, acc, sem):
        c = lax.axis_index("c")
        s = lax.axis_index("s")
        base = s * rows_per_tile
        pltpu.sync_copy(zeros.at[pl.ds(base, rows_per_tile)],
                        acc.at[pl.ds(base, rows_per_tile)])
        plsc.subcore_barrier()

        def run(table, gather_hbm, scatter_hbm, out):
            ebase = s * (chunks * CHUNK)

            def body(i, carry):
                off = ebase + i * CHUNK
                pltpu.sync_copy(gather_hbm.at[pl.ds(off, CHUNK)], idx_g)
                pltpu.sync_copy(scatter_hbm.at[pl.ds(off, CHUNK)], idx_s)
                pltpu.async_copy(table.at[idx_g], rows_v, sem).wait()
                pltpu.sync_copy(rows_v, acc.at[idx_s], add=True)
                return carry

            lax.fori_loop(0, chunks, body, 0)
            plsc.subcore_barrier()
            pltpu.sync_copy(acc.at[pl.ds(base, rows_per_tile)],
                            out.at[pl.ds(base, rows_per_tile)])

        @pl.when(c == 0)
        def _():
            run(z_src, col_idx, row_idx, out_src)

        @pl.when(c == 1)
        def _():
            run(z_dst, row_idx, col_idx, out_dst)

    return agg_kernel


def _mm_body(x_ref, ws_ref, wd_ref, ideg_ref, odeg_ref, zs_ref, zd_ref):
    xb = x_ref[...]
    dn = (((1,), (1,)), ((), ()))
    ys = lax.dot_general(xb, ws_ref[...], dn,
                         preferred_element_type=jnp.float32,
                         precision=lax.Precision.HIGHEST)
    yd = lax.dot_general(xb, wd_ref[...], dn,
                         preferred_element_type=jnp.float32,
                         precision=lax.Precision.HIGHEST)
    ideg = ideg_ref[...]
    odeg = odeg_ref[...]
    iinv = jnp.where(ideg > 0, lax.rsqrt(ideg), 0.0)
    oinv = jnp.where(odeg > 0, lax.rsqrt(odeg), 0.0)
    zs_ref[...] = (ALPHA * iinv) * ys
    zd_ref[...] = ((1.0 - ALPHA) * oinv) * yd


def _fin_body(as_ref, ad_ref, odeg_ref, ideg_ref, bs_ref, bd_ref, out_ref):
    odeg = odeg_ref[...]
    ideg = ideg_ref[...]
    oinv = jnp.where(odeg > 0, lax.rsqrt(odeg), 0.0)
    iinv = jnp.where(ideg > 0, lax.rsqrt(ideg), 0.0)
    bias = ALPHA * bs_ref[...] + (1.0 - ALPHA) * bd_ref[...]
    out_ref[...] = oinv * as_ref[...] + iinv * ad_ref[...] + bias


def kernel(x, edge_index, W_src, b_src, W_dst, b_dst):
    n, d = x.shape
    e = edge_index.shape[1]
    n_pad = _ceil_to(n + 1, 1024)
    e_pad = _ceil_to(e, NTILES * CHUNK * 2)

    row = edge_index[0]
    col = edge_index[1]
    idx_fill = jnp.full((e_pad - e,), n, jnp.int32)
    row_p = jnp.concatenate([row, idx_fill])
    col_p = jnp.concatenate([col, idx_fill])
    x_pad = jnp.pad(x.astype(jnp.float32), ((0, n_pad - n), (0, 0)))

    zeros = jnp.zeros((n_pad, d), jnp.float32)
    zeros16 = jnp.zeros((n_pad, LANES), jnp.float32)
    ones16 = jnp.ones((CHUNK, LANES), jnp.float32)

    odeg16, ideg16 = _make_deg_kernel(n_pad, e_pad)(
        zeros16, ones16, row_p, col_p)
    odeg = odeg16[:, :1]
    ideg = ideg16[:, :1]

    blk = 256
    grid = (n_pad // blk,)
    row_spec = pl.BlockSpec((blk, d), lambda i: (i, 0))
    deg_spec = pl.BlockSpec((blk, 1), lambda i: (i, 0))
    w_spec = pl.BlockSpec((d, d), lambda i: (0, 0))
    z_src, z_dst = pl.pallas_call(
        _mm_body,
        grid=grid,
        in_specs=[row_spec, w_spec, w_spec, deg_spec, deg_spec],
        out_specs=[row_spec, row_spec],
        out_shape=[jax.ShapeDtypeStruct((n_pad, d), jnp.float32)] * 2,
    )(x_pad, W_src, W_dst, ideg, odeg)

    acc_src, acc_dst = _make_agg_kernel(n_pad, e_pad, d)(
        zeros, z_src, z_dst, row_p, col_p)

    bias_spec = pl.BlockSpec((1, d), lambda i: (0, 0))
    out_pad = pl.pallas_call(
        _fin_body,
        grid=grid,
        in_specs=[row_spec, row_spec, deg_spec, deg_spec, bias_spec,
                  bias_spec],
        out_specs=row_spec,
        out_shape=jax.ShapeDtypeStruct((n_pad, d), jnp.float32),
    )(acc_src, acc_dst, odeg, ideg,
      b_src.reshape(1, d), b_dst.reshape(1, d))

    return out_pad[:n]


# speedup vs baseline: 10.1837x; 10.1837x over previous
import functools

import jax
import jax.numpy as jnp
from jax import lax
from jax.experimental import pallas as pl
from jax.experimental.pallas import tpu as pltpu
from jax.experimental.pallas import tpu_sc as plsc

ALPHA = 0.5
LANES = 16
NTILES = 16
CHUNK = 128


def _ceil_to(v, m):
    return -(-v // m) * m


@functools.lru_cache(maxsize=None)
def _make_deg_kernel(n_pad, e_pad):
    chunks = e_pad // (NTILES * CHUNK)
    rows_per_tile = n_pad // NTILES
    mesh = plsc.VectorSubcoreMesh(core_axis_name="c", subcore_axis_name="s")

    @functools.partial(
        pl.kernel,
        out_type=[jax.ShapeDtypeStruct((n_pad, LANES), jnp.float32),
                  jax.ShapeDtypeStruct((n_pad, LANES), jnp.float32)],
        mesh=mesh,
        compiler_params=pltpu.CompilerParams(use_tc_tiling_on_sc=False),
        scratch_types=[
            pltpu.VMEM((CHUNK,), jnp.int32),
            pltpu.VMEM((CHUNK, LANES), jnp.float32),
            pltpu.VMEM_SHARED((n_pad, LANES), jnp.float32),
        ],
    )
    def deg_kernel(zeros16, ones16, row_idx, col_idx, out_deg, in_deg,
                   idx_v, ones_v, acc):
        c = lax.axis_index("c")
        s = lax.axis_index("s")
        base = s * rows_per_tile
        pltpu.sync_copy(zeros16.at[pl.ds(base, rows_per_tile)],
                        acc.at[pl.ds(base, rows_per_tile)])
        pltpu.sync_copy(ones16, ones_v)
        plsc.subcore_barrier()

        def run(idx_hbm, out):
            ebase = s * (chunks * CHUNK)

            def body(i, carry):
                pltpu.sync_copy(idx_hbm.at[pl.ds(ebase + i * CHUNK, CHUNK)],
                                idx_v)
                pltpu.sync_copy(ones_v, acc.at[idx_v], add=True)
                return carry

            lax.fori_loop(0, chunks, body, 0)
            plsc.subcore_barrier()
            pltpu.sync_copy(acc.at[pl.ds(base, rows_per_tile)],
                            out.at[pl.ds(base, rows_per_tile)])

        @pl.when(c == 0)
        def _():
            run(row_idx, out_deg)

        @pl.when(c == 1)
        def _():
            run(col_idx, in_deg)

    return deg_kernel


@functools.lru_cache(maxsize=None)
def _make_agg_kernel(n_pad, e_pad, d):
    chunks = e_pad // (NTILES * CHUNK)
    rows_per_tile = n_pad // NTILES
    mesh = plsc.VectorSubcoreMesh(core_axis_name="c", subcore_axis_name="s")

    @functools.partial(
        pl.kernel,
        out_type=[jax.ShapeDtypeStruct((n_pad, d), jnp.float32),
                  jax.ShapeDtypeStruct((n_pad, d), jnp.float32)],
        mesh=mesh,
        scratch_types=[
            pltpu.VMEM((CHUNK,), jnp.int32),
            pltpu.VMEM((CHUNK,), jnp.int32),
            pltpu.VMEM((CHUNK, d), jnp.float32),
            pltpu.VMEM_SHARED((n_pad, d), jnp.float32),
            pltpu.SemaphoreType.DMA,
        ],
    )
    def agg_kernel(zeros, z_src, z_dst, row_idx, col_idx, out_src, out_dst,
                   idx_g, idx_s, rows_v, acc, sem):
        c = lax.axis_index("c")
        s = lax.axis_index("s")
        base = s * rows_per_tile
        pltpu.sync_copy(zeros.at[pl.ds(base, rows_per_tile)],
                        acc.at[pl.ds(base, rows_per_tile)])
        plsc.subcore_barrier()

        def run(table, gather_hbm, scatter_hbm, out):
            ebase = s * (chunks * CHUNK)

            def body(i, carry):
                off = ebase + i * CHUNK
                pltpu.sync_copy(gather_hbm.at[pl.ds(off, CHUNK)], idx_g)
                pltpu.sync_copy(scatter_hbm.at[pl.ds(off, CHUNK)], idx_s)
                pltpu.async_copy(table.at[idx_g], rows_v, sem).wait()
                pltpu.sync_copy(rows_v, acc.at[idx_s], add=True)
                return carry

            lax.fori_loop(0, chunks, body, 0)
            plsc.subcore_barrier()
            pltpu.sync_copy(acc.at[pl.ds(base, rows_per_tile)],
                            out.at[pl.ds(base, rows_per_tile)])

        @pl.when(c == 0)
        def _():
            run(z_src, col_idx, row_idx, out_src)

        @pl.when(c == 1)
        def _():
            run(z_dst, row_idx, col_idx, out_dst)

    return agg_kernel


def _mm_body(x_ref, ws_ref, wd_ref, ideg_ref, odeg_ref, zs_ref, zd_ref):
    xb = x_ref[...]
    dn = (((1,), (1,)), ((), ()))
    ys = lax.dot_general(xb, ws_ref[...], dn,
                         preferred_element_type=jnp.float32,
                         precision=lax.Precision.HIGHEST)
    yd = lax.dot_general(xb, wd_ref[...], dn,
                         preferred_element_type=jnp.float32,
                         precision=lax.Precision.HIGHEST)
    ideg = ideg_ref[...]
    odeg = odeg_ref[...]
    iinv = jnp.where(ideg > 0, lax.rsqrt(ideg), 0.0)
    oinv = jnp.where(odeg > 0, lax.rsqrt(odeg), 0.0)
    zs_ref[...] = (ALPHA * iinv) * ys
    zd_ref[...] = ((1.0 - ALPHA) * oinv) * yd


def _fin_body(as_ref, ad_ref, odeg_ref, ideg_ref, bs_ref, bd_ref, out_ref):
    odeg = odeg_ref[...]
    ideg = ideg_ref[...]
    oinv = jnp.where(odeg > 0, lax.rsqrt(odeg), 0.0)
    iinv = jnp.where(ideg > 0, lax.rsqrt(ideg), 0.0)
    bias = ALPHA * bs_ref[...] + (1.0 - ALPHA) * bd_ref[...]
    out_ref[...] = oinv * as_ref[...] + iinv * ad_ref[...] + bias


def kernel(x, edge_index, W_src, b_src, W_dst, b_dst):
    n, d = x.shape
    e = edge_index.shape[1]
    n_pad = _ceil_to(n + 1, 1024)
    e_pad = _ceil_to(e, NTILES * CHUNK * 2)

    row = edge_index[0]
    col = edge_index[1]
    idx_fill = jnp.full((e_pad - e,), n, jnp.int32)
    row_p = jnp.concatenate([row, idx_fill])
    col_p = jnp.concatenate([col, idx_fill])
    x_pad = jnp.pad(x.astype(jnp.float32), ((0, n_pad - n), (0, 0)))

    zeros = jnp.zeros((n_pad, d), jnp.float32)
    zeros16 = jnp.zeros((n_pad, LANES), jnp.float32)
    ones16 = jnp.ones((CHUNK, LANES), jnp.float32)

    odeg16, ideg16 = _make_deg_kernel(n_pad, e_pad)(
        zeros16, ones16, row_p, col_p)
    odeg = odeg16[:, :1]
    ideg = ideg16[:, :1]

    blk = 256
    grid = (n_pad // blk,)
    row_spec = pl.BlockSpec((blk, d), lambda i: (i, 0))
    deg_spec = pl.BlockSpec((blk, 1), lambda i: (i, 0))
    w_spec = pl.BlockSpec((d, d), lambda i: (0, 0))
    z_src, z_dst = pl.pallas_call(
        _mm_body,
        grid=grid,
        in_specs=[row_spec, w_spec, w_spec, deg_spec, deg_spec],
        out_specs=[row_spec, row_spec],
        out_shape=[jax.ShapeDtypeStruct((n_pad, d), jnp.float32)] * 2,
    )(x_pad, W_src, W_dst, ideg, odeg)

    acc_src, acc_dst = _make_agg_kernel(n_pad, e_pad, d)(
        zeros, z_src, z_dst, row_p, col_p)

    bias_spec = pl.BlockSpec((1, d), lambda i: (0, 0))
    out_pad = pl.pallas_call(
        _fin_body,
        grid=grid,
        in_specs=[row_spec, row_spec, deg_spec, deg_spec, bias_spec,
                  bias_spec],
        out_specs=row_spec,
        out_shape=jax.ShapeDtypeStruct((n_pad, d), jnp.float32),
    )(acc_src, acc_dst, odeg, ideg,
      b_src.reshape(1, d), b_dst.reshape(1, d))

    return out_pad[:n]


# speedup vs baseline: 10.7970x; 1.0602x over previous
import functools

import jax
import jax.numpy as jnp
from jax import lax
from jax.experimental import pallas as pl
from jax.experimental.pallas import tpu as pltpu
from jax.experimental.pallas import tpu_sc as plsc

ALPHA = 0.5
LANES = 16
NTILES = 16
CHUNK = 128
BLK = 4
BLKE = 2048


def _ceil_to(v, m):
    return -(-v // m) * m


@functools.lru_cache(maxsize=None)
def _make_deg_kernel(n_pad, e_pad):
    ept = e_pad // NTILES
    nblk = ept // BLKE
    seg = n_pad // NTILES
    mesh = plsc.VectorSubcoreMesh(core_axis_name="c", subcore_axis_name="s")
    assert nblk % 2 == 0

    @functools.partial(
        pl.kernel,
        out_type=[jax.ShapeDtypeStruct((n_pad,), jnp.float32),
                  jax.ShapeDtypeStruct((n_pad,), jnp.float32)],
        mesh=mesh,
        compiler_params=pltpu.CompilerParams(
            needs_layout_passes=False, use_tc_tiling_on_sc=False),
        scratch_types=[
            pltpu.VMEM((2, BLKE), jnp.int32),
            pltpu.VMEM((n_pad,), jnp.float32),
            pltpu.VMEM((seg,), jnp.float32),
            pltpu.VMEM((seg,), jnp.float32),
            pltpu.VMEM_SHARED((NTILES, n_pad), jnp.float32),
            pltpu.SemaphoreType.DMA,
            pltpu.SemaphoreType.DMA,
        ],
    )
    def deg_kernel(row_idx, col_idx, out_deg, in_deg,
                   eb, hist, tsum, tin, acc_sh, isem0, isem1):
        c = lax.axis_index("c")
        s = lax.axis_index("s")
        isems = (isem0, isem1)
        zeros16 = jnp.zeros((16,), jnp.float32)
        ones16 = jnp.ones((16,), jnp.float32)

        def zero_hist(i, carry):
            hist[pl.ds(i * 16, 16)] = zeros16
            return carry

        def run(idx3, out):
            lax.fori_loop(0, n_pad // 16, zero_hist, 0)
            pltpu.async_copy(idx3.at[s, 0], eb.at[0], isems[0])
            pltpu.async_copy(idx3.at[s, 1], eb.at[1], isems[1])

            def hist_block(m_val, p):
                def step(kk, carry):
                    for u in range(4):
                        idx = eb[p, pl.ds((kk * 4 + u) * 16, 16)]
                        plsc.addupdate_scatter(hist, [idx], ones16)
                    return carry

                lax.fori_loop(0, BLKE // 64, step, 0)

                @pl.when(m_val < nblk - 2)
                def _():
                    pltpu.async_copy(idx3.at[s, m_val + 2], eb.at[p],
                                     isems[p])

            def outer(t, carry):
                for p in (0, 1):
                    m_val = t * 2 + p
                    pltpu.make_async_copy(idx3.at[s, p], eb.at[p],
                                          isems[p]).wait()
                    hist_block(m_val, p)
                return carry

            lax.fori_loop(0, nblk // 2, outer, 0)
            pltpu.sync_copy(hist, acc_sh.at[s])
            plsc.subcore_barrier()

            base = s * seg
            pltpu.sync_copy(acc_sh.at[0, pl.ds(base, seg)], tsum)
            for k in range(1, NTILES):
                pltpu.sync_copy(acc_sh.at[k, pl.ds(base, seg)], tin)

                def addstep(i, carry):
                    tsum[pl.ds(i * 16, 16)] = (tsum[pl.ds(i * 16, 16)]
                                               + tin[pl.ds(i * 16, 16)])
                    return carry

                lax.fori_loop(0, seg // 16, addstep, 0)
            pltpu.sync_copy(tsum, out.at[pl.ds(base, seg)])

        @pl.when(c == 0)
        def _():
            run(row_idx, out_deg)

        @pl.when(c == 1)
        def _():
            run(col_idx, in_deg)

    return deg_kernel


@functools.lru_cache(maxsize=None)
def _make_agg_kernel(n_pad, e_pad, d):
    chunks = e_pad // (NTILES * CHUNK)
    bpt = chunks // BLK
    t_iters = chunks // 8
    rows_per_tile = n_pad // NTILES
    mesh = plsc.VectorSubcoreMesh(core_axis_name="c", subcore_axis_name="s")
    assert chunks % 8 == 0

    @functools.partial(
        pl.kernel,
        out_type=[jax.ShapeDtypeStruct((n_pad, d), jnp.float32),
                  jax.ShapeDtypeStruct((n_pad, d), jnp.float32)],
        mesh=mesh,
        scratch_types=[
            pltpu.VMEM((2, BLK, CHUNK), jnp.int32),
            pltpu.VMEM((2, BLK, CHUNK), jnp.int32),
            pltpu.VMEM((2, CHUNK, d), jnp.float32),
            pltpu.VMEM_SHARED((n_pad, d), jnp.float32),
            pltpu.SemaphoreType.DMA,
            pltpu.SemaphoreType.DMA,
            pltpu.SemaphoreType.DMA,
            pltpu.SemaphoreType.DMA,
        ],
    )
    def agg_kernel(zeros, z_src, z_dst, row3, col3, out_src, out_dst,
                   ig, isc, rows_v, acc, isem0, isem1, gsem0, gsem1):
        c = lax.axis_index("c")
        s = lax.axis_index("s")
        base = s * rows_per_tile
        isems = (isem0, isem1)
        gsems = (gsem0, gsem1)
        pltpu.sync_copy(zeros.at[pl.ds(base, rows_per_tile)],
                        acc.at[pl.ds(base, rows_per_tile)])
        plsc.subcore_barrier()

        def run(table, g3, s3, out):
            blk0 = s * bpt

            def fire_gather(p, j, b):
                pltpu.async_copy(table.at[ig.at[p, j]], rows_v.at[b],
                                 gsems[b])

            pltpu.sync_copy(g3.at[blk0], ig.at[0])
            pltpu.sync_copy(s3.at[blk0], isc.at[0])
            pltpu.async_copy(g3.at[blk0 + 1], ig.at[1], isems[1])
            pltpu.async_copy(s3.at[blk0 + 1], isc.at[1], isems[1])
            fire_gather(0, 0, 0)
            fire_gather(0, 1, 1)

            def body(t, carry):
                not_last = t < t_iters - 1
                for jj in range(8):
                    p = jj // 4
                    j = jj % 4
                    b = jj % 2
                    if jj == 2:
                        pltpu.make_async_copy(g3.at[blk0], ig.at[1],
                                              isems[1]).wait()
                        pltpu.make_async_copy(s3.at[blk0], isc.at[1],
                                              isems[1]).wait()
                    if jj == 6:
                        @pl.when(not_last)
                        def _():
                            pltpu.make_async_copy(g3.at[blk0], ig.at[0],
                                                  isems[0]).wait()
                            pltpu.make_async_copy(s3.at[blk0], isc.at[0],
                                                  isems[0]).wait()
                    pltpu.make_async_copy(table.at[ig.at[p, j]],
                                          rows_v.at[b], gsems[b]).wait()
                    pltpu.sync_copy(rows_v.at[b], acc.at[isc.at[p, j]],
                                    add=True)
                    if jj == 3:
                        @pl.when(not_last)
                        def _():
                            blk = blk0 + 2 * t + 2
                            pltpu.async_copy(g3.at[blk], ig.at[0], isems[0])
                            pltpu.async_copy(s3.at[blk], isc.at[0], isems[0])
                    if jj == 7:
                        @pl.when(not_last)
                        def _():
                            blk = blk0 + 2 * t + 3
                            pltpu.async_copy(g3.at[blk], ig.at[1], isems[1])
                            pltpu.async_copy(s3.at[blk], isc.at[1], isems[1])
                    pf = ((jj + 2) // 4) % 2
                    jf = (jj + 2) % 4
                    if jj < 6:
                        fire_gather(pf, jf, b)
                    else:
                        @pl.when(not_last)
                        def _():
                            fire_gather(pf, jf, b)
                return carry

            lax.fori_loop(0, t_iters, body, 0)
            plsc.subcore_barrier()
            pltpu.sync_copy(acc.at[pl.ds(base, rows_per_tile)],
                            out.at[pl.ds(base, rows_per_tile)])

        @pl.when(c == 0)
        def _():
            run(z_src, col3, row3, out_src)

        @pl.when(c == 1)
        def _():
            run(z_dst, row3, col3, out_dst)

    return agg_kernel


def _mm_body(x_ref, ws_ref, wd_ref, ideg_ref, odeg_ref, zs_ref, zd_ref):
    xb = x_ref[...]
    dn = (((1,), (1,)), ((), ()))
    ys = lax.dot_general(xb, ws_ref[...], dn,
                         preferred_element_type=jnp.float32,
                         precision=lax.Precision.HIGHEST)
    yd = lax.dot_general(xb, wd_ref[...], dn,
                         preferred_element_type=jnp.float32,
                         precision=lax.Precision.HIGHEST)
    ideg = ideg_ref[...]
    odeg = odeg_ref[...]
    iinv = jnp.where(ideg > 0, lax.rsqrt(ideg), 0.0)
    oinv = jnp.where(odeg > 0, lax.rsqrt(odeg), 0.0)
    zs_ref[...] = (ALPHA * iinv) * ys
    zd_ref[...] = ((1.0 - ALPHA) * oinv) * yd


def _fin_body(as_ref, ad_ref, odeg_ref, ideg_ref, bs_ref, bd_ref, out_ref):
    odeg = odeg_ref[...]
    ideg = ideg_ref[...]
    oinv = jnp.where(odeg > 0, lax.rsqrt(odeg), 0.0)
    iinv = jnp.where(ideg > 0, lax.rsqrt(ideg), 0.0)
    bias = ALPHA * bs_ref[...] + (1.0 - ALPHA) * bd_ref[...]
    out_ref[...] = oinv * as_ref[...] + iinv * ad_ref[...] + bias


def kernel(x, edge_index, W_src, b_src, W_dst, b_dst):
    n, d = x.shape
    e = edge_index.shape[1]
    n_pad = _ceil_to(n + 1, 1024)
    e_pad = _ceil_to(e, NTILES * CHUNK * 8)

    row = edge_index[0]
    col = edge_index[1]
    idx_fill = jnp.full((e_pad - e,), n, jnp.int32)
    row_p = jnp.concatenate([row, idx_fill])
    col_p = jnp.concatenate([col, idx_fill])
    row3 = row_p.reshape(e_pad // (BLK * CHUNK), BLK, CHUNK)
    col3 = col_p.reshape(e_pad // (BLK * CHUNK), BLK, CHUNK)
    rowe = row_p.reshape(NTILES, e_pad // (NTILES * BLKE), BLKE)
    cole = col_p.reshape(NTILES, e_pad // (NTILES * BLKE), BLKE)
    x_pad = jnp.pad(x.astype(jnp.float32), ((0, n_pad - n), (0, 0)))
    zeros = jnp.zeros((n_pad, d), jnp.float32)

    odeg1, ideg1 = _make_deg_kernel(n_pad, e_pad)(rowe, cole)
    odeg = odeg1.reshape(n_pad, 1)
    ideg = ideg1.reshape(n_pad, 1)

    blk = 256
    grid = (n_pad // blk,)
    row_spec = pl.BlockSpec((blk, d), lambda i: (i, 0))
    deg_spec = pl.BlockSpec((blk, 1), lambda i: (i, 0))
    w_spec = pl.BlockSpec((d, d), lambda i: (0, 0))
    z_src, z_dst = pl.pallas_call(
        _mm_body,
        grid=grid,
        in_specs=[row_spec, w_spec, w_spec, deg_spec, deg_spec],
        out_specs=[row_spec, row_spec],
        out_shape=[jax.ShapeDtypeStruct((n_pad, d), jnp.float32)] * 2,
    )(x_pad, W_src, W_dst, ideg, odeg)

    acc_src, acc_dst = _make_agg_kernel(n_pad, e_pad, d)(
        zeros, z_src, z_dst, row3, col3)

    bias_spec = pl.BlockSpec((1, d), lambda i: (0, 0))
    out_pad = pl.pallas_call(
        _fin_body,
        grid=grid,
        in_specs=[row_spec, row_spec, deg_spec, deg_spec, bias_spec,
                  bias_spec],
        out_specs=row_spec,
        out_shape=jax.ShapeDtypeStruct((n_pad, d), jnp.float32),
    )(acc_src, acc_dst, odeg, ideg,
      b_src.reshape(1, d), b_dst.reshape(1, d))

    return out_pad[:n]
